# baseline (speedup 1.0000x reference)
import functools

import jax
import jax.numpy as jnp
from jax import lax
from jax.experimental import pallas as pl
from jax.experimental.pallas import tpu as pltpu
from jax.experimental.pallas import tpu_sc as plsc

_MAXLEN = 200
_D = 64
_BATCH = 4096
_NC = 2
_NS = 16
_NW = _NC * _NS
_BPW = _BATCH // _NW
_NBUF = 4
_LANES = 16
_NK = _BPW // _LANES
_DT = _D // 8
_BT = _BATCH // 128

_mesh = plsc.VectorSubcoreMesh(core_axis_name="c", subcore_axis_name="s")

_SCRATCH = (
    [
        pltpu.VMEM((_MAXLEN // 8, 8, _BPW), jnp.int32),
        pltpu.VMEM((_MAXLEN, _D), jnp.float32),
    ]
    + [pltpu.VMEM((_BPW, _D), jnp.float32)] * _NBUF
    + [pltpu.VMEM((8, 8, 129), jnp.float32)] * _NBUF
    + [pltpu.SemaphoreType.DMA] * (2 * _NBUF)
)


@functools.partial(
    pl.kernel,
    mesh=_mesh,
    compiler_params=pltpu.CompilerParams(
        use_tc_tiling_on_sc=False, needs_layout_passes=False
    ),
    out_type=jax.ShapeDtypeStruct((_MAXLEN, _DT, _BT, 8, 128), jnp.float32),
    scratch_types=_SCRATCH,
)
def _emb_kernel(x_hbm, tok_hbm, pos_hbm, out_hbm, xblk, pos_v,
                rw0, rw1, rw2, rw3, tr0, tr1, tr2, tr3,
                sg0, sg1, sg2, sg3, so0, so1, so2, so3):
    rows = (rw0, rw1, rw2, rw3)
    trans = (tr0, tr1, tr2, tr3)
    sem_g = (sg0, sg1, sg2, sg3)
    sem_o = (so0, so1, so2, so3)
    wid = lax.axis_index("s") * _NC + lax.axis_index("c")
    pltpu.sync_copy(x_hbm.at[pl.ds(0, _MAXLEN // 8), wid], xblk)
    pltpu.sync_copy(pos_hbm, pos_v)

    iota = lax.iota(jnp.int32, _LANES)
    dtv = tuple((iota + (_LANES * v)) // 8 for v in range(_D // _LANES))
    dsv = tuple(lax.rem(iota + (_LANES * v), 8) for v in range(_D // _LANES))

    def issue_gather(l, b):
        pltpu.async_copy(tok_hbm.at[xblk.at[l // 8, lax.rem(l, 8)]],
                         rows[b], sem_g[b])

    def wait_gather(b):
        pltpu.make_async_copy(tok_hbm.at[pl.ds(0, _BPW)], rows[b],
                              sem_g[b]).wait()

    def transpose_add(l, b):
        posv = tuple(pos_v[l, pl.ds(v * _LANES, _LANES)]
                     for v in range(_D // _LANES))

        @plsc.parallel_loop(0, _BPW, unroll=4)
        def body(bl):
            blv = jnp.full((_LANES,), bl, jnp.int32)
            for v in range(_D // _LANES):
                r = rows[b][bl, pl.ds(v * _LANES, _LANES)]
                plsc.store_scatter(trans[b], [dtv[v], dsv[v], blv],
                                   r + posv[v])

    def issue_out(l, b):
        pltpu.async_copy(trans[b].at[pl.ds(0, 8), pl.ds(0, 8), pl.ds(0, 128)],
                         out_hbm.at[l, pl.ds(0, _DT), wid], sem_o[b])

    def wait_out(b):
        pltpu.make_async_copy(
            trans[b].at[pl.ds(0, 8), pl.ds(0, 8), pl.ds(0, 128)],
            out_hbm.at[0, pl.ds(0, _DT), 0], sem_o[b]).wait()

    for l in range(2):
        issue_gather(l, l)

    for b in range(_NBUF):
        wait_gather(b)
        transpose_add(b, b)
        issue_out(b, b)
        b2 = (b + 2) % _NBUF
        issue_gather(b + 2, b2)

    def outer(g, c):
        for b in range(_NBUF):
            l = g * _NBUF + b
            wait_gather(b)
            wait_out(b)
            transpose_add(l, b)
            issue_out(l, b)
            b2 = (b + 2) % _NBUF
            issue_gather(l + 2, b2)
        return c

    lax.fori_loop(1, _MAXLEN // _NBUF - 1, outer, 0)

    for b in range(_NBUF):
        l = _MAXLEN - _NBUF + b
        wait_gather(b)
        wait_out(b)
        transpose_add(l, b)
        issue_out(l, b)
        if b < 2:
            b2 = (b + 2) % _NBUF
            issue_gather(l + 2, b2)
    for b in range(_NBUF):
        wait_out(b)


def kernel(x, token_table, pos_table):
    xq = (jnp.transpose(x.astype(jnp.int32))
          .reshape(_MAXLEN // 8, 8, _BT, 128)
          .transpose(0, 2, 1, 3))
    o5 = _emb_kernel(xq, token_table, pos_table)
    ot = jnp.transpose(o5, (2, 4, 0, 1, 3))
    return ot.reshape(_BATCH, _MAXLEN, _D)


# speedup vs baseline: 1.1497x; 1.1497x over previous
import functools

import jax
import jax.numpy as jnp
from jax import lax
from jax.experimental import pallas as pl
from jax.experimental.pallas import tpu as pltpu
from jax.experimental.pallas import tpu_sc as plsc

_MAXLEN = 200
_D = 64
_BATCH = 4096
_NC = 2
_NS = 16
_NW = _NC * _NS
_BPW = _BATCH // _NW
_NBUF = 4
_LANES = 16
_NK = _BPW // _LANES
_DT = _D // 8
_BT = _BATCH // 128

_mesh = plsc.VectorSubcoreMesh(core_axis_name="c", subcore_axis_name="s")

_SCRATCH = (
    [
        pltpu.VMEM((_MAXLEN // 8, 8, _BPW), jnp.int32),
        pltpu.VMEM((_MAXLEN, _D), jnp.float32),
    ]
    + [pltpu.VMEM((_BPW, _D), jnp.float32)] * _NBUF
    + [pltpu.VMEM((8, 8, 129), jnp.float32)] * _NBUF
    + [pltpu.SemaphoreType.DMA] * (2 * _NBUF)
)


@functools.partial(
    pl.kernel,
    mesh=_mesh,
    compiler_params=pltpu.CompilerParams(
        use_tc_tiling_on_sc=False, needs_layout_passes=False
    ),
    out_type=jax.ShapeDtypeStruct((_MAXLEN, _DT, _BT, 8, 128), jnp.float32),
    scratch_types=_SCRATCH,
)
def _emb_kernel(x_hbm, tok_hbm, pos_hbm, out_hbm, xblk, pos_v,
                rw0, rw1, rw2, rw3, tr0, tr1, tr2, tr3,
                sg0, sg1, sg2, sg3, so0, so1, so2, so3):
    rows = (rw0, rw1, rw2, rw3)
    trans = (tr0, tr1, tr2, tr3)
    sem_g = (sg0, sg1, sg2, sg3)
    sem_o = (so0, so1, so2, so3)
    wid = lax.axis_index("s") * _NC + lax.axis_index("c")
    pltpu.sync_copy(x_hbm.at[pl.ds(0, _MAXLEN // 8), wid], xblk)
    pltpu.sync_copy(pos_hbm, pos_v)

    iota = lax.iota(jnp.int32, _LANES)
    dtv = tuple((iota + (_LANES * v)) // 8 for v in range(_D // _LANES))
    dsv = tuple(lax.rem(iota + (_LANES * v), 8) for v in range(_D // _LANES))

    def issue_gather(l, b):
        pltpu.async_copy(tok_hbm.at[xblk.at[l // 8, lax.rem(l, 8)]],
                         rows[b], sem_g[b])

    def wait_gather(b):
        pltpu.make_async_copy(tok_hbm.at[pl.ds(0, _BPW)], rows[b],
                              sem_g[b]).wait()

    def transpose_add(l, b):
        posv = tuple(pos_v[l, pl.ds(v * _LANES, _LANES)]
                     for v in range(_D // _LANES))

        @plsc.parallel_loop(0, _BPW, unroll=4)
        def body(bl):
            blv = jnp.full((_LANES,), bl, jnp.int32)
            for v in range(_D // _LANES):
                r = rows[b][bl, pl.ds(v * _LANES, _LANES)]
                plsc.store_scatter(trans[b], [dtv[v], dsv[v], blv],
                                   r + posv[v])

    def issue_out(l, b):
        pltpu.async_copy(trans[b].at[pl.ds(0, 8), pl.ds(0, 8), pl.ds(0, 128)],
                         out_hbm.at[l, pl.ds(0, _DT), wid], sem_o[b])

    def wait_out(b):
        pltpu.make_async_copy(
            trans[b].at[pl.ds(0, 8), pl.ds(0, 8), pl.ds(0, 128)],
            out_hbm.at[0, pl.ds(0, _DT), 0], sem_o[b]).wait()

    for l in range(3):
        issue_gather(l, l)

    for b in range(_NBUF):
        wait_gather(b)
        transpose_add(b, b)
        issue_out(b, b)
        issue_gather(b + 3, (b + 3) % _NBUF)

    def outer(g, c):
        for b in range(_NBUF):
            l = g * _NBUF + b
            wait_gather(b)
            wait_out(b)
            transpose_add(l, b)
            issue_out(l, b)
            issue_gather(l + 3, (b + 3) % _NBUF)
        return c

    lax.fori_loop(1, _MAXLEN // _NBUF - 1, outer, 0)

    for b in range(_NBUF):
        l = _MAXLEN - _NBUF + b
        wait_gather(b)
        wait_out(b)
        transpose_add(l, b)
        issue_out(l, b)
        if b < 1:
            issue_gather(l + 3, (b + 3) % _NBUF)
    for b in range(_NBUF):
        wait_out(b)


def kernel(x, token_table, pos_table):
    xq = (jnp.transpose(x.astype(jnp.int32))
          .reshape(_MAXLEN // 8, 8, _BT, 128)
          .transpose(0, 2, 1, 3))
    o5 = _emb_kernel(xq, token_table, pos_table)
    ot = jnp.transpose(o5, (2, 4, 0, 1, 3))
    return ot.reshape(_BATCH, _MAXLEN, _D)


# speedup vs baseline: 1.1783x; 1.0249x over previous
import functools

import jax
import jax.numpy as jnp
from jax import lax
from jax.experimental import pallas as pl
from jax.experimental.pallas import tpu as pltpu
from jax.experimental.pallas import tpu_sc as plsc

_MAXLEN = 200
_D = 64
_BATCH = 4096
_NC = 2
_NS = 16
_NW = _NC * _NS
_BPW = _BATCH // _NW
_NBUF = 5
_LANES = 16
_NK = _BPW // _LANES
_DT = _D // 8
_BT = _BATCH // 128

_mesh = plsc.VectorSubcoreMesh(core_axis_name="c", subcore_axis_name="s")

_SCRATCH = (
    [
        pltpu.VMEM((_MAXLEN // 8, 8, _BPW), jnp.int32),
        pltpu.VMEM((_MAXLEN, _D), jnp.float32),
    ]
    + [pltpu.VMEM((_BPW, _D), jnp.float32)] * _NBUF
    + [pltpu.VMEM((8, 8, 129), jnp.float32)] * _NBUF
    + [pltpu.SemaphoreType.DMA] * (2 * _NBUF)
)


@functools.partial(
    pl.kernel,
    mesh=_mesh,
    compiler_params=pltpu.CompilerParams(
        use_tc_tiling_on_sc=False, needs_layout_passes=False
    ),
    out_type=jax.ShapeDtypeStruct((_MAXLEN, _DT, _BT, 8, 128), jnp.float32),
    scratch_types=_SCRATCH,
)
def _emb_kernel(x_hbm, tok_hbm, pos_hbm, out_hbm, xblk, pos_v,
                rw0, rw1, rw2, rw3, rw4, tr0, tr1, tr2, tr3, tr4,
                sg0, sg1, sg2, sg3, sg4, so0, so1, so2, so3, so4):
    rows = (rw0, rw1, rw2, rw3, rw4)
    trans = (tr0, tr1, tr2, tr3, tr4)
    sem_g = (sg0, sg1, sg2, sg3, sg4)
    sem_o = (so0, so1, so2, so3, so4)
    wid = lax.axis_index("s") * _NC + lax.axis_index("c")
    pltpu.sync_copy(x_hbm.at[pl.ds(0, _MAXLEN // 8), wid], xblk)
    pltpu.sync_copy(pos_hbm, pos_v)

    iota = lax.iota(jnp.int32, _LANES)
    dtv = tuple((iota + (_LANES * v)) // 8 for v in range(_D // _LANES))
    dsv = tuple(lax.rem(iota + (_LANES * v), 8) for v in range(_D // _LANES))

    def issue_gather(l, b):
        pltpu.async_copy(tok_hbm.at[xblk.at[l // 8, lax.rem(l, 8)]],
                         rows[b], sem_g[b])

    def wait_gather(b):
        pltpu.make_async_copy(tok_hbm.at[pl.ds(0, _BPW)], rows[b],
                              sem_g[b]).wait()

    def transpose_add(l, b):
        posv = tuple(pos_v[l, pl.ds(v * _LANES, _LANES)]
                     for v in range(_D // _LANES))

        @plsc.parallel_loop(0, _BPW, unroll=4)
        def body(bl):
            blv = jnp.full((_LANES,), bl, jnp.int32)
            for v in range(_D // _LANES):
                r = rows[b][bl, pl.ds(v * _LANES, _LANES)]
                plsc.store_scatter(trans[b], [dtv[v], dsv[v], blv],
                                   r + posv[v])

    def issue_out(l, b):
        pltpu.async_copy(trans[b].at[pl.ds(0, 8), pl.ds(0, 8), pl.ds(0, 128)],
                         out_hbm.at[l, pl.ds(0, _DT), wid], sem_o[b])

    def wait_out(b):
        pltpu.make_async_copy(
            trans[b].at[pl.ds(0, 8), pl.ds(0, 8), pl.ds(0, 128)],
            out_hbm.at[0, pl.ds(0, _DT), 0], sem_o[b]).wait()

    for l in range(4):
        issue_gather(l, l)

    for b in range(_NBUF):
        wait_gather(b)
        transpose_add(b, b)
        issue_out(b, b)
        issue_gather(b + 4, (b + 4) % _NBUF)

    def outer(g, c):
        for b in range(_NBUF):
            l = g * _NBUF + b
            wait_gather(b)
            wait_out(b)
            transpose_add(l, b)
            issue_out(l, b)
            issue_gather(l + 4, (b + 4) % _NBUF)
        return c

    lax.fori_loop(1, _MAXLEN // _NBUF - 1, outer, 0)

    for b in range(_NBUF):
        l = _MAXLEN - _NBUF + b
        wait_gather(b)
        wait_out(b)
        transpose_add(l, b)
        issue_out(l, b)
        if b < 1:
            issue_gather(l + 4, (b + 4) % _NBUF)
    for b in range(_NBUF):
        wait_out(b)


def kernel(x, token_table, pos_table):
    xq = (jnp.transpose(x.astype(jnp.int32))
          .reshape(_MAXLEN // 8, 8, _BT, 128)
          .transpose(0, 2, 1, 3))
    o5 = _emb_kernel(xq, token_table, pos_table)
    ot = jnp.transpose(o5, (2, 4, 0, 1, 3))
    return ot.reshape(_BATCH, _MAXLEN, _D)
